# Initial kernel scaffold; baseline (speedup 1.0000x reference)
import jax
import jax.numpy as jnp
from jax.experimental import pallas as pl


def kernel(flat_feat, flat_att, segment_ids):
    raise NotImplementedError("write your pallas kernel here")


# speedup vs baseline: 18.9419x; 18.9419x over previous
import functools

import jax
import jax.numpy as jnp
from jax import lax
from jax.experimental import pallas as pl
from jax.experimental.pallas import tpu as pltpu
from jax.experimental.pallas import tpu_sc as plsc

TOTAL = 32768
D = 256
H = 4
B = 16
NC, NS, L = 2, 16, 16
NW = NC * NS
ROWS_W = TOTAL // NW
CHUNK = 128
NCHUNKS = ROWS_W // CHUNK
GROUPS = CHUNK // L
DL = D + L
NSEG = B * H
RSTRIPE = NSEG // NS


def _sc_partial_sums(flat_feat, flat_att_flat, segment_ids):
  mesh = plsc.VectorSubcoreMesh(core_axis_name="c", subcore_axis_name="s")

  @functools.partial(
      pl.kernel,
      out_type=jax.ShapeDtypeStruct((NC, NSEG, DL), jnp.float32),
      mesh=mesh,
      scratch_types=[
          pltpu.VMEM((CHUNK, D), jnp.float32),
          pltpu.VMEM((CHUNK * H,), jnp.float32),
          pltpu.VMEM((CHUNK,), jnp.int32),
          pltpu.VMEM((NSEG, DL), jnp.float32),
          pltpu.VMEM((RSTRIPE, DL), jnp.float32),
          pltpu.VMEM((RSTRIPE, DL), jnp.float32),
          pltpu.VMEM_SHARED((NS, NSEG, DL), jnp.float32),
      ],
  )
  def k(feat_hbm, att_hbm, seg_hbm, out_hbm,
        feat_v, att_v, seg_v, acc_v, red_v, tmp_v, shared):
    cid = lax.axis_index("c")
    sid = lax.axis_index("s")
    wid = cid * NS + sid
    base = wid * ROWS_W

    zero = jnp.zeros((L,), jnp.float32)
    ones = jnp.ones((L,), jnp.float32)

    def zrow(r, carry):
      for j in range(DL // L):
        acc_v[r, pl.ds(j * L, L)] = zero
      return carry
    lax.fori_loop(0, NSEG, zrow, 0)

    def chunk_body(c, carry):
      start = base + c * CHUNK
      pltpu.sync_copy(feat_hbm.at[pl.ds(start, CHUNK)], feat_v)
      pltpu.sync_copy(att_hbm.at[pl.ds(start * H, CHUNK * H)], att_v)
      pltpu.sync_copy(seg_hbm.at[pl.ds(start, CHUNK)], seg_v)

      def group_body(g, gcarry):
        seg16 = seg_v[pl.ds(g * L, L)]
        att4 = [att_v[pl.ds(g * L * H + q * L, L)] for q in range(H)]
        for kk in range(L):
          i = g * L + kk
          s = seg16[kk]
          r0 = s * H
          f = [feat_v[i, pl.ds(j * L, L)] for j in range(D // L)]
          avec = att4[kk // 4]
          for h in range(H):
            a = avec[(kk % 4) * H + h]
            r = r0 + h
            for j in range(D // L):
              plsc.addupdate(acc_v.at[r, pl.ds(j * L, L)], a * f[j])
            plsc.addupdate(acc_v.at[r, pl.ds(D, L)], a * ones)
        return gcarry
      lax.fori_loop(0, GROUPS, group_body, 0)
      return carry
    lax.fori_loop(0, NCHUNKS, chunk_body, 0)

    pltpu.sync_copy(acc_v, shared.at[sid])
    plsc.subcore_barrier()

    rbase = sid * RSTRIPE
    for r in range(RSTRIPE):
      for j in range(DL // L):
        red_v[r, pl.ds(j * L, L)] = zero

    def t_body(t, carry):
      pltpu.sync_copy(shared.at[t, pl.ds(rbase, RSTRIPE)], tmp_v)
      for r in range(RSTRIPE):
        for j in range(DL // L):
          plsc.addupdate(red_v.at[r, pl.ds(j * L, L)],
                         tmp_v[r, pl.ds(j * L, L)])
      return carry
    lax.fori_loop(0, NS, t_body, 0)

    pltpu.sync_copy(red_v, out_hbm.at[cid, pl.ds(rbase, RSTRIPE)])

  return k(flat_feat, flat_att_flat, segment_ids)


def _finish(partials):
  def body(p_ref, avg_ref, ws_ref):
    s = p_ref[0] + p_ref[1]
    ws = s[:, D:D + 1]
    num = s[:, :D]
    safe = jnp.where(ws == 0.0, 1.0, ws)
    avg = jnp.where(ws == 0.0, 0.0, num / safe)
    avg = jnp.where(jnp.isnan(avg), jnp.float32(1e-5), avg)
    avg_ref[...] = avg
    ws_ref[...] = ws

  return pl.pallas_call(
      body,
      out_shape=(jax.ShapeDtypeStruct((NSEG, D), jnp.float32),
                 jax.ShapeDtypeStruct((NSEG, 1), jnp.float32)),
  )(partials)


def kernel(flat_feat, flat_att, segment_ids):
  parts = _sc_partial_sums(flat_feat, jnp.reshape(flat_att, (-1,)),
                           segment_ids)
  avg, ws = _finish(parts)
  return (jnp.reshape(avg, (B, H, D)), jnp.reshape(ws, (B, H)))


# speedup vs baseline: 24.4372x; 1.2901x over previous
import functools

import jax
import jax.numpy as jnp
from jax import lax
from jax.experimental import pallas as pl
from jax.experimental.pallas import tpu as pltpu
from jax.experimental.pallas import tpu_sc as plsc

TOTAL = 32768
D = 256
H = 4
B = 16
NC, NS, L = 2, 16, 16
NW = NC * NS
ROWS_W = TOTAL // NW
CHUNK = 256
NCHUNKS = ROWS_W // CHUNK
GROUPS = CHUNK // L
DL = D + L
NSEG = B * H
RSTRIPE = NSEG // NS
NVH = D // (2 * L)


def _sc_partial_sums(flat_feat, flat_att_flat, segment_ids):
  mesh = plsc.VectorSubcoreMesh(core_axis_name="c", subcore_axis_name="s")

  @functools.partial(
      pl.kernel,
      out_type=jax.ShapeDtypeStruct((NC, NSEG, DL), jnp.float32),
      mesh=mesh,
      scratch_types=[
          pltpu.VMEM((CHUNK, D), jnp.float32),
          pltpu.VMEM((CHUNK * H + L,), jnp.float32),
          pltpu.VMEM((CHUNK,), jnp.int32),
          pltpu.VMEM((NSEG, DL), jnp.float32),
          pltpu.VMEM((RSTRIPE, DL), jnp.float32),
          pltpu.VMEM((RSTRIPE, DL), jnp.float32),
          pltpu.VMEM_SHARED((NS, NSEG, DL), jnp.float32),
      ],
  )
  def k(feat_hbm, att_hbm, seg_hbm, out_hbm,
        feat_v, att_v, seg_v, acc_v, red_v, tmp_v, shared):
    cid = lax.axis_index("c")
    sid = lax.axis_index("s")
    wid = cid * NS + sid
    base = wid * ROWS_W

    zero = jnp.zeros((L,), jnp.float32)
    ones = jnp.ones((L,), jnp.float32)

    def zrow(r, carry):
      for j in range(DL // L):
        acc_v[r, pl.ds(j * L, L)] = zero
      return carry
    lax.fori_loop(0, NSEG, zrow, 0)

    def chunk_body(c, carry):
      start = base + c * CHUNK
      pltpu.sync_copy(feat_hbm.at[pl.ds(start, CHUNK)], feat_v)
      pltpu.sync_copy(att_hbm.at[pl.ds(start * H, CHUNK * H + L)], att_v)
      pltpu.sync_copy(seg_hbm.at[pl.ds(start, CHUNK)], seg_v)

      def group_body(g, gcarry):
        svec = seg_v[pl.ds(g * L, L)]
        single = svec[0] == svec[L - 1]

        @pl.when(single)
        def _():
          s = svec[0]
          for m in range(2):
            acc = [[zero] * NVH for _ in range(H)]
            for kk in range(L):
              i = g * L + kk
              avec = att_v[pl.ds(i * H, L)]
              f = [feat_v[i, pl.ds(m * (D // 2) + j * L, L)]
                   for j in range(NVH)]
              for h in range(H):
                a = avec[h]
                for j in range(NVH):
                  acc[h][j] = acc[h][j] + a * f[j]
                if m == 0:
                  plsc.addupdate(acc_v.at[s * H + h, pl.ds(D, L)], a * ones)
            for h in range(H):
              r = s * H + h
              for j in range(NVH):
                plsc.addupdate(acc_v.at[r, pl.ds(m * (D // 2) + j * L, L)],
                               acc[h][j])

        @pl.when(jnp.logical_not(single))
        def _():
          for kk in range(L):
            i = g * L + kk
            s = svec[kk]
            r0 = s * H
            avec = att_v[pl.ds(i * H, L)]
            f = [feat_v[i, pl.ds(j * L, L)] for j in range(D // L)]
            for h in range(H):
              a = avec[h]
              r = r0 + h
              for j in range(D // L):
                plsc.addupdate(acc_v.at[r, pl.ds(j * L, L)], a * f[j])
              plsc.addupdate(acc_v.at[r, pl.ds(D, L)], a * ones)

        return gcarry
      lax.fori_loop(0, GROUPS, group_body, 0)
      return carry
    lax.fori_loop(0, NCHUNKS, chunk_body, 0)

    pltpu.sync_copy(acc_v, shared.at[sid])
    plsc.subcore_barrier()

    rbase = sid * RSTRIPE
    for r in range(RSTRIPE):
      for j in range(DL // L):
        red_v[r, pl.ds(j * L, L)] = zero

    def t_body(t, carry):
      pltpu.sync_copy(shared.at[t, pl.ds(rbase, RSTRIPE)], tmp_v)
      for r in range(RSTRIPE):
        for j in range(DL // L):
          plsc.addupdate(red_v.at[r, pl.ds(j * L, L)],
                         tmp_v[r, pl.ds(j * L, L)])
      return carry
    lax.fori_loop(0, NS, t_body, 0)

    pltpu.sync_copy(red_v, out_hbm.at[cid, pl.ds(rbase, RSTRIPE)])

  return k(flat_feat, flat_att_flat, segment_ids)


def _finish(partials):
  def body(p_ref, avg_ref, ws_ref):
    s = p_ref[0] + p_ref[1]
    ws = s[:, D:D + 1]
    num = s[:, :D]
    safe = jnp.where(ws == 0.0, 1.0, ws)
    avg = jnp.where(ws == 0.0, 0.0, num / safe)
    avg = jnp.where(jnp.isnan(avg), jnp.float32(1e-5), avg)
    avg_ref[...] = avg
    ws_ref[...] = ws

  return pl.pallas_call(
      body,
      out_shape=(jax.ShapeDtypeStruct((NSEG, D), jnp.float32),
                 jax.ShapeDtypeStruct((NSEG, 1), jnp.float32)),
  )(partials)


def kernel(flat_feat, flat_att, segment_ids):
  att_pad = jnp.concatenate([jnp.reshape(flat_att, (-1,)),
                             jnp.zeros((L,), jnp.float32)])
  parts = _sc_partial_sums(flat_feat, att_pad, segment_ids)
  avg, ws = _finish(parts)
  return (jnp.reshape(avg, (B, H, D)), jnp.reshape(ws, (B, H)))


# speedup vs baseline: 28.1489x; 1.1519x over previous
import functools

import jax
import jax.numpy as jnp
from jax import lax
from jax.experimental import pallas as pl
from jax.experimental.pallas import tpu as pltpu
from jax.experimental.pallas import tpu_sc as plsc

TOTAL = 32768
D = 256
H = 4
B = 16
NC, NS, L = 2, 16, 16
NW = NC * NS
ROWS_W = TOTAL // NW
CHUNK = 128
NCHUNKS = ROWS_W // CHUNK
GROUPS = CHUNK // L
DL = D + L
NSEG = B * H
RSTRIPE = NSEG // NS
NVH = D // (2 * L)


def _sc_partial_sums(flat_feat, flat_att_flat, segment_ids):
  mesh = plsc.VectorSubcoreMesh(core_axis_name="c", subcore_axis_name="s")

  @functools.partial(
      pl.kernel,
      out_type=jax.ShapeDtypeStruct((NC, NSEG, DL), jnp.float32),
      mesh=mesh,
      scratch_types=[
          pltpu.VMEM((2, CHUNK, D), jnp.float32),
          pltpu.VMEM((2 * (CHUNK * H + L),), jnp.float32),
          pltpu.VMEM((2 * CHUNK,), jnp.int32),
          pltpu.VMEM((NSEG, DL), jnp.float32),
          pltpu.VMEM((RSTRIPE, DL), jnp.float32),
          pltpu.VMEM((RSTRIPE, DL), jnp.float32),
          pltpu.VMEM_SHARED((NS, NSEG, DL), jnp.float32),
          pltpu.SemaphoreType.DMA((2,)),
      ],
  )
  def k(feat_hbm, att_hbm, seg_hbm, out_hbm,
        feat2, att2, seg2, acc_v, red_v, tmp_v, shared, sem):
    cid = lax.axis_index("c")
    sid = lax.axis_index("s")
    wid = cid * NS + sid
    base = wid * ROWS_W

    zero = jnp.zeros((L,), jnp.float32)
    ones = jnp.ones((L,), jnp.float32)

    def zrow(r, carry):
      for j in range(DL // L):
        acc_v[r, pl.ds(j * L, L)] = zero
      return carry
    lax.fori_loop(0, NSEG, zrow, 0)

    def issue(c, b):
      start = base + c * CHUNK
      pltpu.async_copy(feat_hbm.at[pl.ds(start, CHUNK)], feat2.at[b],
                       sem.at[b])
      pltpu.async_copy(att_hbm.at[pl.ds(start * H, CHUNK * H + L)],
                       att2.at[pl.ds(b * (CHUNK * H + L), CHUNK * H + L)],
                       sem.at[b])
      pltpu.async_copy(seg_hbm.at[pl.ds(start, CHUNK)],
                       seg2.at[pl.ds(b * CHUNK, CHUNK)], sem.at[b])

    issue(0, 0)

    def chunk_body(c, carry):
      bi = lax.rem(c, 2)
      start = base + c * CHUNK

      @pl.when(c + 1 < NCHUNKS)
      def _():
        issue(c + 1, 1 - bi)

      pltpu.make_async_copy(feat_hbm.at[pl.ds(start, CHUNK)], feat2.at[bi],
                            sem.at[bi]).wait()
      pltpu.make_async_copy(att_hbm.at[pl.ds(start * H, CHUNK * H + L)],
                            att2.at[pl.ds(bi * (CHUNK * H + L),
                                          CHUNK * H + L)],
                            sem.at[bi]).wait()
      pltpu.make_async_copy(seg_hbm.at[pl.ds(start, CHUNK)],
                            seg2.at[pl.ds(bi * CHUNK, CHUNK)],
                            sem.at[bi]).wait()

      def group_body(g, gcarry):
        svec = seg2[pl.ds(bi * CHUNK + g * L, L)]
        single = svec[0] == svec[L - 1]

        @pl.when(single)
        def _():
          s = svec[0]
          for m in range(2):
            acc = [[zero] * NVH for _ in range(H)]
            for kk in range(L):
              i = g * L + kk
              avec = att2[pl.ds(bi * (CHUNK * H + L) + i * H, L)]
              f = [feat2[bi, i, pl.ds(m * (D // 2) + j * L, L)]
                   for j in range(NVH)]
              for h in range(H):
                a = avec[h]
                for j in range(NVH):
                  acc[h][j] = acc[h][j] + a * f[j]
                if m == 0:
                  plsc.addupdate(acc_v.at[s * H + h, pl.ds(D, L)], a * ones)
            for h in range(H):
              r = s * H + h
              for j in range(NVH):
                plsc.addupdate(acc_v.at[r, pl.ds(m * (D // 2) + j * L, L)],
                               acc[h][j])

        @pl.when(jnp.logical_not(single))
        def _():
          for kk in range(L):
            i = g * L + kk
            s = svec[kk]
            r0 = s * H
            avec = att2[pl.ds(bi * (CHUNK * H + L) + i * H, L)]
            f = [feat2[bi, i, pl.ds(j * L, L)] for j in range(D // L)]
            for h in range(H):
              a = avec[h]
              r = r0 + h
              for j in range(D // L):
                plsc.addupdate(acc_v.at[r, pl.ds(j * L, L)], a * f[j])
              plsc.addupdate(acc_v.at[r, pl.ds(D, L)], a * ones)

        return gcarry
      lax.fori_loop(0, GROUPS, group_body, 0)
      return carry
    lax.fori_loop(0, NCHUNKS, chunk_body, 0)

    pltpu.sync_copy(acc_v, shared.at[sid])
    plsc.subcore_barrier()

    rbase = sid * RSTRIPE
    for r in range(RSTRIPE):
      for j in range(DL // L):
        red_v[r, pl.ds(j * L, L)] = zero

    def t_body(t, carry):
      pltpu.sync_copy(shared.at[t, pl.ds(rbase, RSTRIPE)], tmp_v)
      for r in range(RSTRIPE):
        for j in range(DL // L):
          plsc.addupdate(red_v.at[r, pl.ds(j * L, L)],
                         tmp_v[r, pl.ds(j * L, L)])
      return carry
    lax.fori_loop(0, NS, t_body, 0)

    pltpu.sync_copy(red_v, out_hbm.at[cid, pl.ds(rbase, RSTRIPE)])

  return k(flat_feat, flat_att_flat, segment_ids)


def _finish(partials):
  def body(p_ref, avg_ref, ws_ref):
    s = p_ref[0] + p_ref[1]
    ws = s[:, D:D + 1]
    num = s[:, :D]
    safe = jnp.where(ws == 0.0, 1.0, ws)
    avg = jnp.where(ws == 0.0, 0.0, num / safe)
    avg = jnp.where(jnp.isnan(avg), jnp.float32(1e-5), avg)
    avg_ref[...] = avg
    ws_ref[...] = ws

  return pl.pallas_call(
      body,
      out_shape=(jax.ShapeDtypeStruct((NSEG, D), jnp.float32),
                 jax.ShapeDtypeStruct((NSEG, 1), jnp.float32)),
  )(partials)


def kernel(flat_feat, flat_att, segment_ids):
  att_pad = jnp.concatenate([jnp.reshape(flat_att, (-1,)),
                             jnp.zeros((L,), jnp.float32)])
  parts = _sc_partial_sums(flat_feat, att_pad, segment_ids)
  avg, ws = _finish(parts)
  return (jnp.reshape(avg, (B, H, D)), jnp.reshape(ws, (B, H)))


# speedup vs baseline: 28.2323x; 1.0030x over previous
import functools

import jax
import jax.numpy as jnp
from jax import lax
from jax.experimental import pallas as pl
from jax.experimental.pallas import tpu as pltpu
from jax.experimental.pallas import tpu_sc as plsc

TOTAL = 32768
D = 256
H = 4
B = 16
NC, NS, L = 2, 16, 16
NW = NC * NS
ROWS_W = TOTAL // NW
CHUNK = 128
NCHUNKS = ROWS_W // CHUNK
GROUPS = CHUNK // L
DL = D + L
NSEG = B * H
RSTRIPE = NSEG // NS
NVH = D // (2 * L)


def _sc_partial_sums(flat_feat, flat_att_flat, segment_ids):
  mesh = plsc.VectorSubcoreMesh(core_axis_name="c", subcore_axis_name="s")

  @functools.partial(
      pl.kernel,
      out_type=jax.ShapeDtypeStruct((NC, NSEG, DL), jnp.float32),
      mesh=mesh,
      scratch_types=[
          pltpu.VMEM((2, CHUNK, D), jnp.float32),
          pltpu.VMEM((2 * (CHUNK * H + L),), jnp.float32),
          pltpu.VMEM((2 * CHUNK,), jnp.int32),
          pltpu.VMEM((NSEG, DL), jnp.float32),
          pltpu.VMEM((RSTRIPE, DL), jnp.float32),
          pltpu.VMEM((RSTRIPE, DL), jnp.float32),
          pltpu.VMEM_SHARED((NS, NSEG, DL), jnp.float32),
          pltpu.SemaphoreType.DMA((2,)),
      ],
  )
  def k(feat_hbm, att_hbm, seg_hbm, out_hbm,
        feat2, att2, seg2, acc_v, red_v, tmp_v, shared, sem):
    cid = lax.axis_index("c")
    sid = lax.axis_index("s")
    wid = cid * NS + sid
    base = wid * ROWS_W

    zero = jnp.zeros((L,), jnp.float32)
    ones = jnp.ones((L,), jnp.float32)

    def zrow(r, carry):
      for j in range(DL // L):
        acc_v[r, pl.ds(j * L, L)] = zero
      return carry
    lax.fori_loop(0, NSEG, zrow, 0)

    def issue(c, b):
      start = base + c * CHUNK
      pltpu.async_copy(feat_hbm.at[pl.ds(start, CHUNK)], feat2.at[b],
                       sem.at[b])
      pltpu.async_copy(att_hbm.at[pl.ds(start * H, CHUNK * H)],
                       att2.at[pl.ds(b * (CHUNK * H + L), CHUNK * H)],
                       sem.at[b])
      pltpu.async_copy(seg_hbm.at[pl.ds(start, CHUNK)],
                       seg2.at[pl.ds(b * CHUNK, CHUNK)], sem.at[b])

    issue(0, 0)

    def chunk_body(c, carry):
      bi = lax.rem(c, 2)
      start = base + c * CHUNK

      @pl.when(c + 1 < NCHUNKS)
      def _():
        issue(c + 1, 1 - bi)

      pltpu.make_async_copy(feat_hbm.at[pl.ds(start, CHUNK)], feat2.at[bi],
                            sem.at[bi]).wait()
      pltpu.make_async_copy(att_hbm.at[pl.ds(start * H, CHUNK * H)],
                            att2.at[pl.ds(bi * (CHUNK * H + L), CHUNK * H)],
                            sem.at[bi]).wait()
      pltpu.make_async_copy(seg_hbm.at[pl.ds(start, CHUNK)],
                            seg2.at[pl.ds(bi * CHUNK, CHUNK)],
                            sem.at[bi]).wait()

      def group_body(g, gcarry):
        svec = seg2[pl.ds(bi * CHUNK + g * L, L)]
        single = svec[0] == svec[L - 1]

        @pl.when(single)
        def _():
          s = svec[0]
          for m in range(2):
            acc = [[zero] * NVH for _ in range(H)]
            for kk in range(L):
              i = g * L + kk
              avec = att2[pl.ds(bi * (CHUNK * H + L) + i * H, L)]
              f = [feat2[bi, i, pl.ds(m * (D // 2) + j * L, L)]
                   for j in range(NVH)]
              for h in range(H):
                a = avec[h]
                for j in range(NVH):
                  acc[h][j] = acc[h][j] + a * f[j]
                if m == 0:
                  plsc.addupdate(acc_v.at[s * H + h, pl.ds(D, L)], a * ones)
            for h in range(H):
              r = s * H + h
              for j in range(NVH):
                plsc.addupdate(acc_v.at[r, pl.ds(m * (D // 2) + j * L, L)],
                               acc[h][j])

        @pl.when(jnp.logical_not(single))
        def _():
          for kk in range(L):
            i = g * L + kk
            s = svec[kk]
            r0 = s * H
            avec = att2[pl.ds(bi * (CHUNK * H + L) + i * H, L)]
            f = [feat2[bi, i, pl.ds(j * L, L)] for j in range(D // L)]
            for h in range(H):
              a = avec[h]
              r = r0 + h
              for j in range(D // L):
                plsc.addupdate(acc_v.at[r, pl.ds(j * L, L)], a * f[j])
              plsc.addupdate(acc_v.at[r, pl.ds(D, L)], a * ones)

        return gcarry
      lax.fori_loop(0, GROUPS, group_body, 0)
      return carry
    lax.fori_loop(0, NCHUNKS, chunk_body, 0)

    pltpu.sync_copy(acc_v, shared.at[sid])
    plsc.subcore_barrier()

    rbase = sid * RSTRIPE
    for r in range(RSTRIPE):
      for j in range(DL // L):
        red_v[r, pl.ds(j * L, L)] = zero

    def t_body(t, carry):
      pltpu.sync_copy(shared.at[t, pl.ds(rbase, RSTRIPE)], tmp_v)
      for r in range(RSTRIPE):
        for j in range(DL // L):
          plsc.addupdate(red_v.at[r, pl.ds(j * L, L)],
                         tmp_v[r, pl.ds(j * L, L)])
      return carry
    lax.fori_loop(0, NS, t_body, 0)

    pltpu.sync_copy(red_v, out_hbm.at[cid, pl.ds(rbase, RSTRIPE)])

  return k(flat_feat, flat_att_flat, segment_ids)


def _finish(partials):
  def body(p_ref, avg_ref, ws_ref):
    s = p_ref[0] + p_ref[1]
    ws = s[:, D:D + 1]
    num = s[:, :D]
    safe = jnp.where(ws == 0.0, 1.0, ws)
    avg = jnp.where(ws == 0.0, 0.0, num / safe)
    avg = jnp.where(jnp.isnan(avg), jnp.float32(1e-5), avg)
    avg_ref[...] = avg
    ws_ref[...] = ws

  return pl.pallas_call(
      body,
      out_shape=(jax.ShapeDtypeStruct((NSEG, D), jnp.float32),
                 jax.ShapeDtypeStruct((NSEG, 1), jnp.float32)),
  )(partials)


def kernel(flat_feat, flat_att, segment_ids):
  parts = _sc_partial_sums(flat_feat, jnp.reshape(flat_att, (-1,)),
                           segment_ids)
  avg, ws = _finish(parts)
  return (jnp.reshape(avg, (B, H, D)), jnp.reshape(ws, (B, H)))
